# baseline (speedup 1.0000x reference)
import jax
import jax.numpy as jnp
from jax import lax
from jax.experimental import pallas as pl
from jax.experimental.pallas import tpu as pltpu
from jax.experimental.pallas import tpu_sc as plsc

B = 16384
D = 64
NU = 1000000
NM = 100000
T = 8
NC = 2
NS = 16
L = 16
NW = NC * NS
BPW = B // NW
NG = BPW // L


def _fac_body(uid_hbm, mid_hbm, users_hbm, movies_hbm, ub_hbm, mb_hbm,
              out_hbm,
              uidx_v, midx_v, uidx_s, midx_s, ubuf, mbuf, ubb, mbb, out_v,
              sem):
    c = lax.axis_index("c")
    s = lax.axis_index("s")
    wid = s * NC + c

    pltpu.sync_copy(uid_hbm.at[wid], uidx_v)
    pltpu.sync_copy(mid_hbm.at[wid], midx_v)
    pass


    lane = lax.iota(jnp.int32, L)

    def chunk(g, carry):
        uvec = uidx_v[pl.ds(g * L, L)]
        mvec = midx_v[pl.ds(g * L, L)]
        us = lax.bitwise_and(uvec, 7)
        ms = lax.bitwise_and(mvec, 7)
        utv = lax.shift_right_logical(uvec, 3)
        mtv = lax.shift_right_logical(mvec, 3)

        cps = []
        for i in range(L):
            ut = utv[i]
            mt = mtv[i]
            cps.append(pltpu.async_copy(users_hbm.at[ut], ubuf.at[i], sem))
            cps.append(pltpu.async_copy(movies_hbm.at[mt], mbuf.at[i], sem))
            cps.append(pltpu.async_copy(ub_hbm.at[ut], ubb.at[i], sem))
            cps.append(pltpu.async_copy(mb_hbm.at[mt], mbb.at[i], sem))
        for cp in cps:
            cp.wait()

        zero = jnp.zeros((L,), jnp.int32)
        acc = (plsc.load_gather(ubb, [lane, us, zero])
               + plsc.load_gather(mbb, [lane, ms, zero]))
        for k in range(D):
            kv = jnp.full((L,), k, jnp.int32)
            u = plsc.load_gather(ubuf, [lane, us, kv])
            m = plsc.load_gather(mbuf, [lane, ms, kv])
            acc = acc + u * m
        out_v[pl.ds(g * L, L)] = acc
        return carry

    lax.fori_loop(0, NG, chunk, 0)

    pltpu.sync_copy(out_v, out_hbm.at[pl.ds(wid * BPW, BPW)])


def kernel(user_ids, movie_ids, users, movies, user_bias, movie_bias):
    uid = user_ids.astype(jnp.int32).reshape(NW, BPW)
    mid = movie_ids.astype(jnp.int32).reshape(NW, BPW)
    users_t = users.reshape(NU // T, T, D)
    movies_t = movies.reshape(NM // T, T, D)
    ub_t = user_bias.reshape(NU // T, T, 1)
    mb_t = movie_bias.reshape(NM // T, T, 1)

    mesh = plsc.VectorSubcoreMesh(core_axis_name="c", subcore_axis_name="s")
    fn = pl.kernel(
        _fac_body,
        out_type=jax.ShapeDtypeStruct((B,), jnp.float32),
        mesh=mesh,
        compiler_params=pltpu.CompilerParams(
            needs_layout_passes=False, use_tc_tiling_on_sc=True),
        scratch_types=[
            pltpu.VMEM((BPW,), jnp.int32),
            pltpu.VMEM((BPW,), jnp.int32),
            pltpu.SMEM((BPW,), jnp.int32),
            pltpu.SMEM((BPW,), jnp.int32),
            pltpu.VMEM((L, T, D), jnp.float32),
            pltpu.VMEM((L, T, D), jnp.float32),
            pltpu.VMEM((L, T, 1), jnp.float32),
            pltpu.VMEM((L, T, 1), jnp.float32),
            pltpu.VMEM((BPW,), jnp.float32),
            pltpu.SemaphoreType.DMA,
        ],
    )
    return fn(uid, mid, users_t, movies_t, ub_t, mb_t)


# speedup vs baseline: 1.4941x; 1.4941x over previous
import jax
import jax.numpy as jnp
from jax import lax
from jax.experimental import pallas as pl
from jax.experimental.pallas import tpu as pltpu
from jax.experimental.pallas import tpu_sc as plsc

B = 16384
D = 64
NC = 2
NS = 16
L = 16
NW = NC * NS
BPW = B // NW
NG = BPW // L


def _fac_body(uid_hbm, mid_hbm, users_hbm, movies_hbm, ub_hbm, mb_hbm,
              out_hbm,
              uidx_v, midx_v, ubuf, mbuf, ubb, mbb, out_v, sem):
    c = lax.axis_index("c")
    s = lax.axis_index("s")
    wid = s * NC + c

    pltpu.sync_copy(uid_hbm.at[wid], uidx_v)
    pltpu.sync_copy(mid_hbm.at[wid], midx_v)

    lane = lax.iota(jnp.int32, L)

    def chunk(g, carry):
        uvec = uidx_v[pl.ds(g * L, L)]
        mvec = midx_v[pl.ds(g * L, L)]

        cps = []
        for i in range(L):
            ui = uvec[i]
            mi = mvec[i]
            cps.append(pltpu.async_copy(users_hbm.at[ui], ubuf.at[i], sem))
            cps.append(pltpu.async_copy(movies_hbm.at[mi], mbuf.at[i], sem))
            cps.append(pltpu.async_copy(ub_hbm.at[ui], ubb.at[i], sem))
            cps.append(pltpu.async_copy(mb_hbm.at[mi], mbb.at[i], sem))
        for cp in cps:
            cp.wait()

        zero = jnp.zeros((L,), jnp.int32)
        acc = (plsc.load_gather(ubb, [lane, zero])
               + plsc.load_gather(mbb, [lane, zero]))
        for k in range(D):
            kv = jnp.full((L,), k, jnp.int32)
            u = plsc.load_gather(ubuf, [lane, kv])
            m = plsc.load_gather(mbuf, [lane, kv])
            acc = acc + u * m
        out_v[pl.ds(g * L, L)] = acc
        return carry

    lax.fori_loop(0, NG, chunk, 0)

    pltpu.sync_copy(out_v, out_hbm.at[pl.ds(wid * BPW, BPW)])


def kernel(user_ids, movie_ids, users, movies, user_bias, movie_bias):
    uid = user_ids.astype(jnp.int32).reshape(NW, BPW)
    mid = movie_ids.astype(jnp.int32).reshape(NW, BPW)

    mesh = plsc.VectorSubcoreMesh(core_axis_name="c", subcore_axis_name="s")
    fn = pl.kernel(
        _fac_body,
        out_type=jax.ShapeDtypeStruct((B,), jnp.float32),
        mesh=mesh,
        compiler_params=pltpu.CompilerParams(
            needs_layout_passes=False, use_tc_tiling_on_sc=True),
        scratch_types=[
            pltpu.VMEM((BPW,), jnp.int32),
            pltpu.VMEM((BPW,), jnp.int32),
            pltpu.VMEM((L, D), jnp.float32),
            pltpu.VMEM((L, D), jnp.float32),
            pltpu.VMEM((L, 1), jnp.float32),
            pltpu.VMEM((L, 1), jnp.float32),
            pltpu.VMEM((BPW,), jnp.float32),
            pltpu.SemaphoreType.DMA,
        ],
    )
    return fn(uid, mid, users, movies, user_bias, movie_bias)


# speedup vs baseline: 1.5172x; 1.0155x over previous
import jax
import jax.numpy as jnp
from jax import lax
from jax.experimental import pallas as pl
from jax.experimental.pallas import tpu as pltpu
from jax.experimental.pallas import tpu_sc as plsc

B = 16384
D = 64
NC = 2
NS = 16
L = 16
NW = NC * NS
BPW = B // NW
CH = 32
NG = BPW // CH


def _fac_body(uid_hbm, mid_hbm, users_hbm, movies_hbm, ub_hbm, mb_hbm,
              out_hbm,
              uidx_v, midx_v, ubuf, mbuf, ubb, mbb, out_v, sem):
    c = lax.axis_index("c")
    s = lax.axis_index("s")
    wid = s * NC + c

    pltpu.sync_copy(uid_hbm.at[wid], uidx_v)
    pltpu.sync_copy(mid_hbm.at[wid], midx_v)

    lane = lax.iota(jnp.int32, L)

    def chunk(g, carry):
        vecs = []
        for h in range(CH // L):
            vecs.append((uidx_v[pl.ds(g * CH + h * L, L)],
                         midx_v[pl.ds(g * CH + h * L, L)]))

        cps = []
        for h, (uvec, mvec) in enumerate(vecs):
            for i in range(L):
                p = h * L + i
                ui = uvec[i]
                mi = mvec[i]
                cps.append(pltpu.async_copy(users_hbm.at[ui], ubuf.at[p], sem))
                cps.append(pltpu.async_copy(movies_hbm.at[mi], mbuf.at[p], sem))
                cps.append(pltpu.async_copy(ub_hbm.at[ui], ubb.at[p], sem))
                cps.append(pltpu.async_copy(mb_hbm.at[mi], mbb.at[p], sem))
        for cp in cps:
            cp.wait()

        zero = jnp.zeros((L,), jnp.int32)
        for h in range(CH // L):
            pv = h * L + lane
            acc = (plsc.load_gather(ubb, [pv, zero])
                   + plsc.load_gather(mbb, [pv, zero]))
            for k in range(D):
                kv = jnp.full((L,), k, jnp.int32)
                u = plsc.load_gather(ubuf, [pv, kv])
                m = plsc.load_gather(mbuf, [pv, kv])
                acc = acc + u * m
            out_v[pl.ds(g * CH + h * L, L)] = acc
        return carry

    lax.fori_loop(0, NG, chunk, 0)

    pltpu.sync_copy(out_v, out_hbm.at[pl.ds(wid * BPW, BPW)])


def kernel(user_ids, movie_ids, users, movies, user_bias, movie_bias):
    uid = user_ids.astype(jnp.int32).reshape(NW, BPW)
    mid = movie_ids.astype(jnp.int32).reshape(NW, BPW)

    mesh = plsc.VectorSubcoreMesh(core_axis_name="c", subcore_axis_name="s")
    fn = pl.kernel(
        _fac_body,
        out_type=jax.ShapeDtypeStruct((B,), jnp.float32),
        mesh=mesh,
        compiler_params=pltpu.CompilerParams(
            needs_layout_passes=False, use_tc_tiling_on_sc=True),
        scratch_types=[
            pltpu.VMEM((BPW,), jnp.int32),
            pltpu.VMEM((BPW,), jnp.int32),
            pltpu.VMEM((CH, D), jnp.float32),
            pltpu.VMEM((CH, D), jnp.float32),
            pltpu.VMEM((CH, 1), jnp.float32),
            pltpu.VMEM((CH, 1), jnp.float32),
            pltpu.VMEM((BPW,), jnp.float32),
            pltpu.SemaphoreType.DMA,
        ],
    )
    return fn(uid, mid, users, movies, user_bias, movie_bias)
